# baseline (speedup 1.0000x reference)
import jax
import jax.numpy as jnp
from jax import lax
from jax.experimental import pallas as pl
from jax.experimental.pallas import tpu as pltpu
from jax.experimental.pallas import tpu_sc as plsc

NC = 2
NS = 16
NW = NC * NS
L = 16
NUM_SEG = 64
SEG_PER_W = NUM_SEG // NW
N_ROWS = 50000
D = 256
NJ = D // L
CH = 128
U = 4

TCB = 2000
S = 24000
NTCB = (N_ROWS - S) // TCB
TCB0 = S // TCB


def _body(x_hbm, bounds_hbm, out_hbm, bounds_v, buf0_v, buf1_v, acc_v,
          sem0, sem1):
    cid = lax.axis_index("c")
    sid = lax.axis_index("s")
    wid = sid * NC + cid

    pltpu.sync_copy(bounds_hbm, bounds_v)

    def start(a8, c, buf):
        base = a8 + c * CH
        clamped = pl.multiple_of(jnp.minimum(base, N_ROWS - CH), 8)
        sem = sem0 if buf is buf0_v else sem1
        pltpu.make_async_copy(x_hbm.at[pl.ds(clamped, CH)], buf, sem).start()

    def wait(buf):
        sem = sem0 if buf is buf0_v else sem1
        pltpu.make_async_copy(x_hbm.at[pl.ds(0, CH)], buf, sem).wait()

    zero = jnp.zeros((L,), jnp.float32)

    for t in range(SEG_PER_W):
        segid = wid + NW * t
        bv = bounds_v[pl.ds(segid, L)]
        a0 = bv[0]
        b1 = bv[1]

        a8 = lax.div(a0, 8) * 8
        nch = lax.div(b1 - a8 + (CH - 1), CH)
        npairs = lax.div(nch + 1, 2)

        def compute(c, buf, acc, a8=a8, a0=a0, b1=b1):
            base = a8 + c * CH
            clamped = jnp.minimum(base, N_ROWS - CH)
            lo = jnp.maximum(base, a0) - clamped
            hi = jnp.maximum(jnp.minimum(b1, clamped + CH) - clamped, lo)

            def row_body(r, acc):
                return tuple(acc[j] + buf[r, pl.ds(j * L, L)]
                             for j in range(NJ))

            n4 = lax.div(hi - lo, U)

            def body4(i, acc):
                r0 = lo + i * U
                for u in range(U):
                    r = r0 + u
                    acc = tuple(acc[j] + buf[r, pl.ds(j * L, L)]
                                for j in range(NJ))
                return acc

            acc = lax.fori_loop(0, n4, body4, acc)
            return lax.fori_loop(lo + n4 * U, hi, row_body, acc)

        start(a8, 0, buf0_v)
        acc = tuple(zero for _ in range(NJ))

        def pair_body(g, acc, a8=a8, compute=compute):
            c0 = 2 * g
            start(a8, c0 + 1, buf1_v)
            wait(buf0_v)
            acc = compute(c0, buf0_v, acc)
            start(a8, c0 + 2, buf0_v)
            wait(buf1_v)
            acc = compute(c0 + 1, buf1_v, acc)
            return acc

        acc = lax.fori_loop(0, npairs, pair_body, acc)
        wait(buf0_v)

        for j in range(NJ):
            acc_v[pl.ds(j * L, L)] = acc[j]
        pltpu.sync_copy(acc_v, out_hbm.at[pl.ds(segid * D, D)])


def _sc_pool(x, bounds):
    mesh = plsc.VectorSubcoreMesh(core_axis_name="c", subcore_axis_name="s",
                                  num_cores=NC, num_subcores=NS)
    return pl.kernel(
        _body,
        out_type=jax.ShapeDtypeStruct((NUM_SEG * D,), jnp.float32),
        mesh=mesh,
        scratch_types=[
            pltpu.VMEM((80,), jnp.int32),
            pltpu.VMEM((CH, D), jnp.float32),
            pltpu.VMEM((CH, D), jnp.float32),
            pltpu.VMEM((D,), jnp.float32),
            pltpu.SemaphoreType.DMA,
            pltpu.SemaphoreType.DMA,
        ],
    )(x, bounds)


def _tc_body(batch_ref, x_ref, out_ref):
    b = pl.program_id(0)
    seg = lax.broadcasted_iota(jnp.int32, (NUM_SEG, TCB), 0)
    onehot = (seg == batch_ref[0]).astype(jnp.float32)
    p = jnp.dot(onehot, x_ref[...], preferred_element_type=jnp.float32)

    @pl.when(b == 0)
    def _():
        out_ref[...] = jnp.zeros_like(out_ref)

    out_ref[...] += p


_tc_pool = pl.pallas_call(
    _tc_body,
    grid=(NTCB,),
    in_specs=[
        pl.BlockSpec((1, 1, TCB), lambda b: (b + TCB0, 0, 0)),
        pl.BlockSpec((TCB, D), lambda b: (b + TCB0, 0)),
    ],
    out_specs=pl.BlockSpec((NUM_SEG, D), lambda b: (0, 0)),
    out_shape=jax.ShapeDtypeStruct((NUM_SEG, D), jnp.float32),
)


def kernel(x, batch):
    seg = jnp.arange(NUM_SEG, dtype=batch.dtype)
    counts = jnp.sum((batch[:, None] == seg[None, :]).astype(jnp.int32),
                     axis=0)
    bounds = jnp.concatenate(
        [jnp.zeros((1,), jnp.int32), jnp.cumsum(counts),
         jnp.full((15,), N_ROWS, jnp.int32)]).astype(jnp.int32)

    batch3 = batch.astype(jnp.int32).reshape(N_ROWS // TCB, 1, TCB)
    tc_sums = _tc_pool(batch3, x)
    sc_sums = _sc_pool(x, jnp.minimum(bounds, S)).reshape(NUM_SEG, D)

    cf = counts.astype(jnp.float32)
    inv = jnp.where(cf > 0.0, 1.0 / jnp.maximum(cf, 1.0), 0.0)
    return (sc_sums + tc_sums) * inv[:, None]


# speedup vs baseline: 1.0502x; 1.0502x over previous
import jax
import jax.numpy as jnp
from jax import lax
from jax.experimental import pallas as pl
from jax.experimental.pallas import tpu as pltpu
from jax.experimental.pallas import tpu_sc as plsc

NC = 2
NS = 16
NW = NC * NS
L = 16
NUM_SEG = 64
SEG_PER_W = NUM_SEG // NW
N_ROWS = 50000
D = 256
NJ = D // L
CH = 128
U = 4

TCB = 2000
S = 24000
NTCB = (N_ROWS - S) // TCB
TCB0 = S // TCB


def _body(x_hbm, bounds_hbm, out_hbm, bounds_v, buf0_v, buf1_v, acc_v,
          sem0, sem1):
    cid = lax.axis_index("c")
    sid = lax.axis_index("s")
    wid = sid * NC + cid

    pltpu.sync_copy(bounds_hbm, bounds_v)

    def start(a8, c, buf):
        base = a8 + c * CH
        clamped = pl.multiple_of(jnp.minimum(base, N_ROWS - CH), 8)
        sem = sem0 if buf is buf0_v else sem1
        pltpu.make_async_copy(x_hbm.at[pl.ds(clamped, CH)], buf, sem).start()

    def wait(buf):
        sem = sem0 if buf is buf0_v else sem1
        pltpu.make_async_copy(x_hbm.at[pl.ds(0, CH)], buf, sem).wait()

    zero = jnp.zeros((L,), jnp.float32)

    for t in range(SEG_PER_W):
        segid = wid + NW * t
        bv = bounds_v[pl.ds(segid, L)]
        a0 = bv[0]
        b1 = bv[1]

        a8 = lax.div(a0, 8) * 8
        nch = lax.div(b1 - a8 + (CH - 1), CH)
        npairs = lax.div(nch + 1, 2)

        def compute(c, buf, acc, a8=a8, a0=a0, b1=b1):
            base = a8 + c * CH
            clamped = jnp.minimum(base, N_ROWS - CH)
            lo = jnp.maximum(base, a0) - clamped
            hi = jnp.maximum(jnp.minimum(b1, clamped + CH) - clamped, lo)

            def row_body(r, acc):
                return tuple(acc[j] + buf[r, pl.ds(j * L, L)]
                             for j in range(NJ))

            n4 = lax.div(hi - lo, U)

            def body4(i, acc):
                r0 = lo + i * U
                for u in range(U):
                    r = r0 + u
                    acc = tuple(acc[j] + buf[r, pl.ds(j * L, L)]
                                for j in range(NJ))
                return acc

            acc = lax.fori_loop(0, n4, body4, acc)
            return lax.fori_loop(lo + n4 * U, hi, row_body, acc)

        @pl.when(npairs > 0)
        def _():
            start(a8, 0, buf0_v)
        acc = tuple(zero for _ in range(NJ))

        def pair_body(g, acc, a8=a8, compute=compute):
            c0 = 2 * g
            start(a8, c0 + 1, buf1_v)
            wait(buf0_v)
            acc = compute(c0, buf0_v, acc)
            start(a8, c0 + 2, buf0_v)
            wait(buf1_v)
            acc = compute(c0 + 1, buf1_v, acc)
            return acc

        acc = lax.fori_loop(0, npairs, pair_body, acc)

        @pl.when(npairs > 0)
        def _():
            wait(buf0_v)

        for j in range(NJ):
            acc_v[pl.ds(j * L, L)] = acc[j]
        pltpu.sync_copy(acc_v, out_hbm.at[pl.ds(segid * D, D)])


def _sc_pool(x, bounds):
    mesh = plsc.VectorSubcoreMesh(core_axis_name="c", subcore_axis_name="s",
                                  num_cores=NC, num_subcores=NS)
    return pl.kernel(
        _body,
        out_type=jax.ShapeDtypeStruct((NUM_SEG * D,), jnp.float32),
        mesh=mesh,
        scratch_types=[
            pltpu.VMEM((80,), jnp.int32),
            pltpu.VMEM((CH, D), jnp.float32),
            pltpu.VMEM((CH, D), jnp.float32),
            pltpu.VMEM((D,), jnp.float32),
            pltpu.SemaphoreType.DMA,
            pltpu.SemaphoreType.DMA,
        ],
    )(x, bounds)


def _tc_body(batch_ref, x_ref, out_ref):
    b = pl.program_id(0)
    seg = lax.broadcasted_iota(jnp.int32, (NUM_SEG, TCB), 0)
    onehot = (seg == batch_ref[0]).astype(jnp.float32)
    p = jnp.dot(onehot, x_ref[...], preferred_element_type=jnp.float32)

    @pl.when(b == 0)
    def _():
        out_ref[...] = jnp.zeros_like(out_ref)

    out_ref[...] += p


_tc_pool = pl.pallas_call(
    _tc_body,
    grid=(NTCB,),
    in_specs=[
        pl.BlockSpec((1, 1, TCB), lambda b: (b + TCB0, 0, 0)),
        pl.BlockSpec((TCB, D), lambda b: (b + TCB0, 0)),
    ],
    out_specs=pl.BlockSpec((NUM_SEG, D), lambda b: (0, 0)),
    out_shape=jax.ShapeDtypeStruct((NUM_SEG, D), jnp.float32),
)


def kernel(x, batch):
    seg = jnp.arange(NUM_SEG, dtype=batch.dtype)
    counts = jnp.sum((batch[:, None] == seg[None, :]).astype(jnp.int32),
                     axis=0)
    bounds = jnp.concatenate(
        [jnp.zeros((1,), jnp.int32), jnp.cumsum(counts),
         jnp.full((15,), N_ROWS, jnp.int32)]).astype(jnp.int32)

    batch3 = batch.astype(jnp.int32).reshape(N_ROWS // TCB, 1, TCB)
    tc_sums = _tc_pool(batch3, x)
    sc_sums = _sc_pool(x, jnp.minimum(bounds, S)).reshape(NUM_SEG, D)

    cf = counts.astype(jnp.float32)
    inv = jnp.where(cf > 0.0, 1.0 / jnp.maximum(cf, 1.0), 0.0)
    return (sc_sums + tc_sums) * inv[:, None]


# speedup vs baseline: 1.0737x; 1.0223x over previous
import jax
import jax.numpy as jnp
from jax import lax
from jax.experimental import pallas as pl
from jax.experimental.pallas import tpu as pltpu
from jax.experimental.pallas import tpu_sc as plsc

NC = 2
NS = 16
NW = NC * NS
L = 16
NUM_SEG = 64
SEG_PER_W = NUM_SEG // NW
N_ROWS = 50000
D = 256
NJ = D // L
CH = 128
U = 4

TCB = 2000
S = 20000
NTCB = (N_ROWS - S) // TCB
TCB0 = S // TCB


def _body(x_hbm, bounds_hbm, out_hbm, bounds_v, buf0_v, buf1_v, acc_v,
          sem0, sem1):
    cid = lax.axis_index("c")
    sid = lax.axis_index("s")
    wid = sid * NC + cid

    pltpu.sync_copy(bounds_hbm, bounds_v)

    def start(a8, c, buf):
        base = a8 + c * CH
        clamped = pl.multiple_of(jnp.minimum(base, N_ROWS - CH), 8)
        sem = sem0 if buf is buf0_v else sem1
        pltpu.make_async_copy(x_hbm.at[pl.ds(clamped, CH)], buf, sem).start()

    def wait(buf):
        sem = sem0 if buf is buf0_v else sem1
        pltpu.make_async_copy(x_hbm.at[pl.ds(0, CH)], buf, sem).wait()

    zero = jnp.zeros((L,), jnp.float32)

    for t in range(SEG_PER_W):
        segid = wid + NW * t
        bv = bounds_v[pl.ds(segid, L)]
        a0 = bv[0]
        b1 = bv[1]

        a8 = lax.div(a0, 8) * 8
        nch = lax.div(b1 - a8 + (CH - 1), CH)
        npairs = lax.div(nch + 1, 2)

        def compute(c, buf, acc, a8=a8, a0=a0, b1=b1):
            base = a8 + c * CH
            clamped = jnp.minimum(base, N_ROWS - CH)
            lo = jnp.maximum(base, a0) - clamped
            hi = jnp.maximum(jnp.minimum(b1, clamped + CH) - clamped, lo)

            def row_body(r, acc):
                return tuple(acc[j] + buf[r, pl.ds(j * L, L)]
                             for j in range(NJ))

            n4 = lax.div(hi - lo, U)

            def body4(i, acc):
                r0 = lo + i * U
                for u in range(U):
                    r = r0 + u
                    acc = tuple(acc[j] + buf[r, pl.ds(j * L, L)]
                                for j in range(NJ))
                return acc

            acc = lax.fori_loop(0, n4, body4, acc)
            return lax.fori_loop(lo + n4 * U, hi, row_body, acc)

        @pl.when(npairs > 0)
        def _():
            start(a8, 0, buf0_v)
        acc = tuple(zero for _ in range(NJ))

        def pair_body(g, acc, a8=a8, compute=compute):
            c0 = 2 * g
            start(a8, c0 + 1, buf1_v)
            wait(buf0_v)
            acc = compute(c0, buf0_v, acc)
            start(a8, c0 + 2, buf0_v)
            wait(buf1_v)
            acc = compute(c0 + 1, buf1_v, acc)
            return acc

        acc = lax.fori_loop(0, npairs, pair_body, acc)

        @pl.when(npairs > 0)
        def _():
            wait(buf0_v)

        for j in range(NJ):
            acc_v[pl.ds(j * L, L)] = acc[j]
        pltpu.sync_copy(acc_v, out_hbm.at[pl.ds(segid * D, D)])


def _sc_pool(x, bounds):
    mesh = plsc.VectorSubcoreMesh(core_axis_name="c", subcore_axis_name="s",
                                  num_cores=NC, num_subcores=NS)
    return pl.kernel(
        _body,
        out_type=jax.ShapeDtypeStruct((NUM_SEG * D,), jnp.float32),
        mesh=mesh,
        scratch_types=[
            pltpu.VMEM((80,), jnp.int32),
            pltpu.VMEM((CH, D), jnp.float32),
            pltpu.VMEM((CH, D), jnp.float32),
            pltpu.VMEM((D,), jnp.float32),
            pltpu.SemaphoreType.DMA,
            pltpu.SemaphoreType.DMA,
        ],
    )(x, bounds)


def _tc_body(batch_ref, x_ref, out_ref):
    b = pl.program_id(0)
    seg = lax.broadcasted_iota(jnp.int32, (NUM_SEG, TCB), 0)
    onehot = (seg == batch_ref[0]).astype(jnp.float32)
    p = jnp.dot(onehot, x_ref[...], preferred_element_type=jnp.float32)

    @pl.when(b == 0)
    def _():
        out_ref[...] = jnp.zeros_like(out_ref)

    out_ref[...] += p


_tc_pool = pl.pallas_call(
    _tc_body,
    grid=(NTCB,),
    in_specs=[
        pl.BlockSpec((1, 1, TCB), lambda b: (b + TCB0, 0, 0)),
        pl.BlockSpec((TCB, D), lambda b: (b + TCB0, 0)),
    ],
    out_specs=pl.BlockSpec((NUM_SEG, D), lambda b: (0, 0)),
    out_shape=jax.ShapeDtypeStruct((NUM_SEG, D), jnp.float32),
)


def kernel(x, batch):
    seg = jnp.arange(NUM_SEG, dtype=batch.dtype)
    counts = jnp.sum((batch[:, None] == seg[None, :]).astype(jnp.int32),
                     axis=0)
    bounds = jnp.concatenate(
        [jnp.zeros((1,), jnp.int32), jnp.cumsum(counts),
         jnp.full((15,), N_ROWS, jnp.int32)]).astype(jnp.int32)

    batch3 = batch.astype(jnp.int32).reshape(N_ROWS // TCB, 1, TCB)
    tc_sums = _tc_pool(batch3, x)
    sc_sums = _sc_pool(x, jnp.minimum(bounds, S)).reshape(NUM_SEG, D)

    cf = counts.astype(jnp.float32)
    inv = jnp.where(cf > 0.0, 1.0 / jnp.maximum(cf, 1.0), 0.0)
    return (sc_sums + tc_sums) * inv[:, None]


# speedup vs baseline: 1.0757x; 1.0019x over previous
import jax
import jax.numpy as jnp
from jax import lax
from jax.experimental import pallas as pl
from jax.experimental.pallas import tpu as pltpu
from jax.experimental.pallas import tpu_sc as plsc

NC = 2
NS = 16
NW = NC * NS
L = 16
NUM_SEG = 64
SEG_PER_W = NUM_SEG // NW
N_ROWS = 50000
D = 256
NJ = D // L
CH = 128
U = 4

TCB = 2000
S = 16000
NTCB = (N_ROWS - S) // TCB
TCB0 = S // TCB


def _body(x_hbm, bounds_hbm, out_hbm, bounds_v, buf0_v, buf1_v, acc_v,
          sem0, sem1):
    cid = lax.axis_index("c")
    sid = lax.axis_index("s")
    wid = sid * NC + cid

    pltpu.sync_copy(bounds_hbm, bounds_v)

    def start(a8, c, buf):
        base = a8 + c * CH
        clamped = pl.multiple_of(jnp.minimum(base, N_ROWS - CH), 8)
        sem = sem0 if buf is buf0_v else sem1
        pltpu.make_async_copy(x_hbm.at[pl.ds(clamped, CH)], buf, sem).start()

    def wait(buf):
        sem = sem0 if buf is buf0_v else sem1
        pltpu.make_async_copy(x_hbm.at[pl.ds(0, CH)], buf, sem).wait()

    zero = jnp.zeros((L,), jnp.float32)

    for t in range(SEG_PER_W):
        segid = wid + NW * t
        bv = bounds_v[pl.ds(segid, L)]
        a0 = bv[0]
        b1 = bv[1]

        a8 = lax.div(a0, 8) * 8
        nch = lax.div(b1 - a8 + (CH - 1), CH)
        npairs = lax.div(nch + 1, 2)

        def compute(c, buf, acc, a8=a8, a0=a0, b1=b1):
            base = a8 + c * CH
            clamped = jnp.minimum(base, N_ROWS - CH)
            lo = jnp.maximum(base, a0) - clamped
            hi = jnp.maximum(jnp.minimum(b1, clamped + CH) - clamped, lo)

            def row_body(r, acc):
                return tuple(acc[j] + buf[r, pl.ds(j * L, L)]
                             for j in range(NJ))

            n4 = lax.div(hi - lo, U)

            def body4(i, acc):
                r0 = lo + i * U
                for u in range(U):
                    r = r0 + u
                    acc = tuple(acc[j] + buf[r, pl.ds(j * L, L)]
                                for j in range(NJ))
                return acc

            acc = lax.fori_loop(0, n4, body4, acc)
            return lax.fori_loop(lo + n4 * U, hi, row_body, acc)

        @pl.when(npairs > 0)
        def _():
            start(a8, 0, buf0_v)
        acc = tuple(zero for _ in range(NJ))

        def pair_body(g, acc, a8=a8, compute=compute):
            c0 = 2 * g
            start(a8, c0 + 1, buf1_v)
            wait(buf0_v)
            acc = compute(c0, buf0_v, acc)
            start(a8, c0 + 2, buf0_v)
            wait(buf1_v)
            acc = compute(c0 + 1, buf1_v, acc)
            return acc

        acc = lax.fori_loop(0, npairs, pair_body, acc)

        @pl.when(npairs > 0)
        def _():
            wait(buf0_v)

        for j in range(NJ):
            acc_v[pl.ds(j * L, L)] = acc[j]
        pltpu.sync_copy(acc_v, out_hbm.at[pl.ds(segid * D, D)])


def _sc_pool(x, bounds):
    mesh = plsc.VectorSubcoreMesh(core_axis_name="c", subcore_axis_name="s",
                                  num_cores=NC, num_subcores=NS)
    return pl.kernel(
        _body,
        out_type=jax.ShapeDtypeStruct((NUM_SEG * D,), jnp.float32),
        mesh=mesh,
        scratch_types=[
            pltpu.VMEM((80,), jnp.int32),
            pltpu.VMEM((CH, D), jnp.float32),
            pltpu.VMEM((CH, D), jnp.float32),
            pltpu.VMEM((D,), jnp.float32),
            pltpu.SemaphoreType.DMA,
            pltpu.SemaphoreType.DMA,
        ],
    )(x, bounds)


def _tc_body(batch_ref, x_ref, out_ref):
    b = pl.program_id(0)
    seg = lax.broadcasted_iota(jnp.int32, (NUM_SEG, TCB), 0)
    onehot = (seg == batch_ref[0]).astype(jnp.float32)
    p = jnp.dot(onehot, x_ref[...], preferred_element_type=jnp.float32)

    @pl.when(b == 0)
    def _():
        out_ref[...] = jnp.zeros_like(out_ref)

    out_ref[...] += p


_tc_pool = pl.pallas_call(
    _tc_body,
    grid=(NTCB,),
    in_specs=[
        pl.BlockSpec((1, 1, TCB), lambda b: (b + TCB0, 0, 0)),
        pl.BlockSpec((TCB, D), lambda b: (b + TCB0, 0)),
    ],
    out_specs=pl.BlockSpec((NUM_SEG, D), lambda b: (0, 0)),
    out_shape=jax.ShapeDtypeStruct((NUM_SEG, D), jnp.float32),
)


def kernel(x, batch):
    seg = jnp.arange(NUM_SEG, dtype=batch.dtype)
    counts = jnp.sum((batch[:, None] == seg[None, :]).astype(jnp.int32),
                     axis=0)
    bounds = jnp.concatenate(
        [jnp.zeros((1,), jnp.int32), jnp.cumsum(counts),
         jnp.full((15,), N_ROWS, jnp.int32)]).astype(jnp.int32)

    batch3 = batch.astype(jnp.int32).reshape(N_ROWS // TCB, 1, TCB)
    tc_sums = _tc_pool(batch3, x)
    sc_sums = _sc_pool(x, jnp.minimum(bounds, S)).reshape(NUM_SEG, D)

    cf = counts.astype(jnp.float32)
    inv = jnp.where(cf > 0.0, 1.0 / jnp.maximum(cf, 1.0), 0.0)
    return (sc_sums + tc_sums) * inv[:, None]
